# baseline (speedup 1.0000x reference)
import functools

import jax
import jax.numpy as jnp
import numpy as np
from jax.experimental import pallas as pl
from jax.experimental.pallas import tpu as pltpu

N = 10000
E = 320000
D = 128
DE = 16
DELTA = 2.5

_NT = 2000
_ET = 8000


def _mm_kernel(x_ref, w_ref, o_ref):
    o_ref[...] = jnp.dot(x_ref[...], w_ref[...], preferred_element_type=jnp.float32)


def _mm_bias_kernel(x_ref, w_ref, b_ref, o_ref):
    o_ref[...] = jnp.dot(x_ref[...], w_ref[...], preferred_element_type=jnp.float32) + b_ref[...]


def _node_proj(node_feat, w_nd):
    return pl.pallas_call(
        _mm_kernel,
        grid=(N // _NT,),
        in_specs=[
            pl.BlockSpec((_NT, D), lambda i: (i, 0)),
            pl.BlockSpec((D, 2 * D), lambda i: (0, 0)),
        ],
        out_specs=pl.BlockSpec((_NT, 2 * D), lambda i: (i, 0)),
        out_shape=jax.ShapeDtypeStruct((N, 2 * D), jnp.float32),
    )(node_feat, w_nd)


def _edge_proj(edge_feat, w_e, b_m):
    return pl.pallas_call(
        _mm_bias_kernel,
        grid=(E // _ET,),
        in_specs=[
            pl.BlockSpec((_ET, DE), lambda i: (i, 0)),
            pl.BlockSpec((DE, D), lambda i: (0, 0)),
            pl.BlockSpec((1, D), lambda i: (0, 0)),
        ],
        out_specs=pl.BlockSpec((_ET, D), lambda i: (i, 0)),
        out_shape=jax.ShapeDtypeStruct((E, D), jnp.float32),
    )(edge_feat, w_e, b_m)


def _post_kernel(nf_ref, s_ref, mx_ref, mn_ref, sq_ref, deg_ref, wu_ref, bu_ref,
                 y_ref, s1_ref, s2_ref, acc_ref):
    i = pl.program_id(0)
    deg = deg_ref[...]
    degc = jnp.maximum(deg, 1.0)
    has = deg > 0
    mean = s_ref[...] / degc
    mx = jnp.where(has, mx_ref[...], 0.0)
    mn = jnp.where(has, mn_ref[...], 0.0)
    sq = sq_ref[...] / degc
    std = jnp.sqrt(jax.nn.relu(sq - mean * mean) + 1e-30)
    logd = jnp.log(deg + 1.0)
    a = logd * (1.0 / DELTA)
    t = DELTA / (logd + 1e-30)
    h = jnp.concatenate(
        [nf_ref[...], mean, mx, mn, std,
         mean * a, mx * a, mn * a, std * a,
         mean * t, mx * t, mn * t, std * t], axis=1)
    y = (jnp.dot(h, wu_ref[...], preferred_element_type=jnp.float32)
         + bu_ref[...]) * (1.0 / np.sqrt(N))
    y_ref[...] = y

    @pl.when(i == 0)
    def _():
        acc_ref[...] = jnp.zeros_like(acc_ref)

    acc_ref[0:1, :] += jnp.sum(y, axis=0, keepdims=True)
    acc_ref[1:2, :] += jnp.sum(y * y, axis=0, keepdims=True)

    @pl.when(i == pl.num_programs(0) - 1)
    def _():
        s1_ref[...] = acc_ref[0:1, :]
        s2_ref[...] = acc_ref[1:2, :]


def _post(node_feat, s, mx, mn, sq, deg2d, w_u, b_u):
    return pl.pallas_call(
        _post_kernel,
        grid=(N // _NT,),
        in_specs=[
            pl.BlockSpec((_NT, D), lambda i: (i, 0)),
            pl.BlockSpec((_NT, D), lambda i: (i, 0)),
            pl.BlockSpec((_NT, D), lambda i: (i, 0)),
            pl.BlockSpec((_NT, D), lambda i: (i, 0)),
            pl.BlockSpec((_NT, D), lambda i: (i, 0)),
            pl.BlockSpec((_NT, 1), lambda i: (i, 0)),
            pl.BlockSpec((13 * D, D), lambda i: (0, 0)),
            pl.BlockSpec((1, D), lambda i: (0, 0)),
        ],
        out_specs=[
            pl.BlockSpec((_NT, D), lambda i: (i, 0)),
            pl.BlockSpec((1, D), lambda i: (0, 0)),
            pl.BlockSpec((1, D), lambda i: (0, 0)),
        ],
        out_shape=[
            jax.ShapeDtypeStruct((N, D), jnp.float32),
            jax.ShapeDtypeStruct((1, D), jnp.float32),
            jax.ShapeDtypeStruct((1, D), jnp.float32),
        ],
        scratch_shapes=[pltpu.VMEM((2, D), jnp.float32)],
    )(node_feat, s, mx, mn, sq, deg2d, w_u, b_u)


def _bn_kernel(y_ref, s1_ref, s2_ref, g_ref, b_ref, o_ref):
    mu = s1_ref[...] * (1.0 / N)
    var = s2_ref[...] * (1.0 / N) - mu * mu
    o_ref[...] = (y_ref[...] - mu) * jax.lax.rsqrt(var + 1e-5) * g_ref[...] + b_ref[...]


def _bn(y, s1, s2, gamma, beta):
    return pl.pallas_call(
        _bn_kernel,
        grid=(N // _NT,),
        in_specs=[
            pl.BlockSpec((_NT, D), lambda i: (i, 0)),
            pl.BlockSpec((1, D), lambda i: (0, 0)),
            pl.BlockSpec((1, D), lambda i: (0, 0)),
            pl.BlockSpec((1, D), lambda i: (0, 0)),
            pl.BlockSpec((1, D), lambda i: (0, 0)),
        ],
        out_specs=pl.BlockSpec((_NT, D), lambda i: (i, 0)),
        out_shape=jax.ShapeDtypeStruct((N, D), jnp.float32),
    )(y, s1, s2, gamma, beta)


def kernel(node_feat, edge_index, edge_feat, W_M, b_M, W_U, b_U, gamma, beta):
    src = edge_index[0]
    dst = edge_index[1]
    w_nd = jnp.concatenate([W_M[:D, :], W_M[D : 2 * D, :]], axis=1)
    p = _node_proj(node_feat, w_nd)
    p_src, p_dst = p[:, :D], p[:, D:]
    ep = _edge_proj(edge_feat, W_M[2 * D :, :], b_M.reshape(1, D))

    msg = p_src[src] + p_dst[dst] + ep

    ones_e = jnp.ones((E,), dtype=jnp.float32)
    deg = jax.ops.segment_sum(ones_e, dst, num_segments=N)
    s = jax.ops.segment_sum(msg, dst, num_segments=N)
    mx = jax.ops.segment_max(msg, dst, num_segments=N)
    mx = jnp.where((deg > 0)[:, None], mx, 0.0)
    mn = jax.ops.segment_min(msg, dst, num_segments=N)
    mn = jnp.where((deg > 0)[:, None], mn, 0.0)
    sq = jax.ops.segment_sum(msg * msg, dst, num_segments=N)

    y, s1, s2 = _post(node_feat, s, mx, mn, sq, deg.reshape(N, 1),
                      W_U, b_U.reshape(1, D))
    return _bn(y, s1, s2, gamma.reshape(1, D), beta.reshape(1, D))
